# Initial kernel scaffold; baseline (speedup 1.0000x reference)
import functools

import jax
import jax.numpy as jnp
from jax import lax
from jax.experimental import pallas as pl
from jax.experimental.pallas import tpu as pltpu
from jax.experimental.pallas import tpu_sc as plsc

N, E, D, DE, H = 10000, 320000, 128, 16, 128

NW = 32
CHUNK = 128
EDGES_PER_TILE = 10112
EP = NW * EDGES_PER_TILE
NCHUNKS = EDGES_PER_TILE // CHUNK
NP = 10240
RPT = NP // 16
DUMMY = N

BN = 1024
BE = 4096

_mesh = plsc.VectorSubcoreMesh(core_axis_name="c", subcore_axis_name="s")


@functools.partial(
    pl.kernel,
    out_type=jax.ShapeDtypeStruct((2, NP, H), jnp.float32),
    mesh=_mesh,
    scratch_types=[
        pltpu.VMEM((CHUNK,), jnp.int32),
        pltpu.VMEM((CHUNK,), jnp.int32),
        pltpu.VMEM((CHUNK, H), jnp.float32),
        pltpu.VMEM_SHARED((NP, H), jnp.float32),
        pltpu.SemaphoreType.DMA,
    ],
)
def _sc_gather_scatter(h_hbm, src_hbm, dst_hbm, zer_hbm, out_hbm,
                       src_v, dst_v, rows_v, acc_sh, sem):
    cid = lax.axis_index("c")
    sid = lax.axis_index("s")
    wid = sid * 2 + cid
    rbase = sid * RPT
    pltpu.sync_copy(zer_hbm.at[pl.ds(rbase, RPT)], acc_sh.at[pl.ds(rbase, RPT)])
    plsc.subcore_barrier()

    ebase = wid * EDGES_PER_TILE

    def body(j, carry):
        off = ebase + j * CHUNK
        pltpu.sync_copy(src_hbm.at[pl.ds(off, CHUNK)], src_v)
        pltpu.sync_copy(dst_hbm.at[pl.ds(off, CHUNK)], dst_v)
        pltpu.async_copy(h_hbm.at[src_v], rows_v, sem).wait()
        pltpu.sync_copy(rows_v, acc_sh.at[dst_v], add=True)
        return carry

    lax.fori_loop(0, NCHUNKS, body, 0)
    plsc.subcore_barrier()
    pltpu.sync_copy(acc_sh.at[pl.ds(rbase, RPT)],
                    out_hbm.at[cid].at[pl.ds(rbase, RPT)])


@functools.partial(
    pl.kernel,
    out_type=(jax.ShapeDtypeStruct((2, NP, H), jnp.float32),
              jax.ShapeDtypeStruct((2, NP, 16), jnp.float32)),
    mesh=_mesh,
    scratch_types=[
        pltpu.VMEM((CHUNK,), jnp.int32),
        pltpu.VMEM((CHUNK, H), jnp.float32),
        pltpu.VMEM((CHUNK, 16), jnp.float32),
        pltpu.VMEM_SHARED((NP, H), jnp.float32),
        pltpu.VMEM_SHARED((NP, 16), jnp.float32),
        pltpu.SemaphoreType.DMA,
    ],
)
def _sc_scatter_rows(e_hbm, dst_hbm, zer_hbm, zer16_hbm, ones_hbm,
                     qout_hbm, dout_hbm,
                     dst_v, rows_v, ones_v, acc_sh, deg_sh, sem):
    cid = lax.axis_index("c")
    sid = lax.axis_index("s")
    wid = sid * 2 + cid
    rbase = sid * RPT
    pltpu.sync_copy(zer_hbm.at[pl.ds(rbase, RPT)], acc_sh.at[pl.ds(rbase, RPT)])
    pltpu.sync_copy(zer16_hbm.at[pl.ds(rbase, RPT)], deg_sh.at[pl.ds(rbase, RPT)])
    pltpu.sync_copy(ones_hbm, ones_v)
    plsc.subcore_barrier()

    ebase = wid * EDGES_PER_TILE

    def body(j, carry):
        off = ebase + j * CHUNK
        pltpu.sync_copy(dst_hbm.at[pl.ds(off, CHUNK)], dst_v)
        pltpu.sync_copy(e_hbm.at[pl.ds(off, CHUNK)], rows_v)
        pltpu.sync_copy(rows_v, acc_sh.at[dst_v], add=True)
        pltpu.sync_copy(ones_v, deg_sh.at[dst_v], add=True)
        return carry

    lax.fori_loop(0, NCHUNKS, body, 0)
    plsc.subcore_barrier()
    pltpu.sync_copy(acc_sh.at[pl.ds(rbase, RPT)],
                    qout_hbm.at[cid].at[pl.ds(rbase, RPT)])
    pltpu.sync_copy(deg_sh.at[pl.ds(rbase, RPT)],
                    dout_hbm.at[cid].at[pl.ds(rbase, RPT)])


def _edge_body(ef_ref, WeT_ref, be_ref, e_ref):
    e_ref[...] = (jnp.dot(ef_ref[...], WeT_ref[...],
                          preferred_element_type=jnp.float32) + be_ref[...])


def _init_body(nf_ref, WnT_ref, bn_ref, q0_ref, q1_ref, d0_ref, d1_ref,
               Wm2T_ref, bm_ref, h0_ref, eb_ref):
    h0_ref[...] = (jnp.dot(nf_ref[...], WnT_ref[...],
                           preferred_element_type=jnp.float32) + bn_ref[...])
    deg = d0_ref[:, 0:1] + d1_ref[:, 0:1]
    eb_ref[...] = (jnp.dot(q0_ref[...] + q1_ref[...], Wm2T_ref[...],
                           preferred_element_type=jnp.float32)
                   + deg * bm_ref[...])


def _gru_body(p0_ref, p1_ref, eb_ref, h_ref, Wm1T_ref, W_ihT_ref, b_ih_ref,
              W_hhT_ref, b_hh_ref, hn_ref):
    agg = (jnp.dot(p0_ref[...] + p1_ref[...], Wm1T_ref[...],
                   preferred_element_type=jnp.float32) + eb_ref[...])
    gi = jnp.dot(agg, W_ihT_ref[...],
                 preferred_element_type=jnp.float32) + b_ih_ref[...]
    gh = jnp.dot(h_ref[...], W_hhT_ref[...],
                 preferred_element_type=jnp.float32) + b_hh_ref[...]
    r = jax.nn.sigmoid(gi[:, :H] + gh[:, :H])
    z = jax.nn.sigmoid(gi[:, H:2 * H] + gh[:, H:2 * H])
    n = jnp.tanh(gi[:, 2 * H:] + r * gh[:, 2 * H:])
    hn_ref[...] = (1.0 - z) * n + z * h_ref[...]


def _row_spec(b, w):
    return pl.BlockSpec((b, w), lambda i: (i, 0))


def _full_spec(r, c):
    return pl.BlockSpec((r, c), lambda i: (0, 0))


_edge_call = pl.pallas_call(
    _edge_body,
    grid=(EP // BE,),
    in_specs=[_row_spec(BE, DE), _full_spec(DE, H), _full_spec(1, H)],
    out_specs=_row_spec(BE, H),
    out_shape=jax.ShapeDtypeStruct((EP, H), jnp.float32),
)

_init_call = pl.pallas_call(
    _init_body,
    grid=(NP // BN,),
    in_specs=[_row_spec(BN, D), _full_spec(D, H), _full_spec(1, H),
              _row_spec(BN, H), _row_spec(BN, H),
              _row_spec(BN, 16), _row_spec(BN, 16),
              _full_spec(H, H), _full_spec(1, H)],
    out_specs=[_row_spec(BN, H), _row_spec(BN, H)],
    out_shape=[jax.ShapeDtypeStruct((NP, H), jnp.float32),
               jax.ShapeDtypeStruct((NP, H), jnp.float32)],
)

_gru_call = pl.pallas_call(
    _gru_body,
    grid=(NP // BN,),
    in_specs=[_row_spec(BN, H), _row_spec(BN, H), _row_spec(BN, H),
              _row_spec(BN, H),
              _full_spec(H, H), _full_spec(H, 3 * H), _full_spec(1, 3 * H),
              _full_spec(H, 3 * H), _full_spec(1, 3 * H)],
    out_specs=_row_spec(BN, H),
    out_shape=jax.ShapeDtypeStruct((NP, H), jnp.float32),
)


def kernel(node_feat, edge_index, edge_feat, Wn, bn, We, be, Wm, bm,
           W_ih, b_ih, W_hh, b_hh):
    src = edge_index[0]
    dst = edge_index[1]
    pad = EP - E
    src_p = jnp.concatenate([src, jnp.zeros((pad,), jnp.int32)])
    dst_p = jnp.concatenate([dst, jnp.full((pad,), DUMMY, jnp.int32)])
    ef_p = jnp.concatenate([edge_feat, jnp.zeros((pad, DE), jnp.float32)])
    nf_p = jnp.concatenate([node_feat, jnp.zeros((NP - N, D), jnp.float32)])

    zer = jnp.zeros((NP, H), jnp.float32)
    zer16 = jnp.zeros((NP, 16), jnp.float32)
    ones = jnp.ones((CHUNK, 16), jnp.float32)

    e_p = _edge_call(ef_p, We.T, be.reshape(1, H))
    q, dcnt = _sc_scatter_rows(e_p, dst_p, zer, zer16, ones)
    h, ebias = _init_call(nf_p, Wn.T, bn.reshape(1, H), q[0], q[1],
                          dcnt[0], dcnt[1], Wm[:, H:].T, bm.reshape(1, H))
    for _ in range(3):
        p = _sc_gather_scatter(h, src_p, dst_p, zer)
        h = _gru_call(p[0], p[1], ebias, h, Wm[:, :H].T, W_ih.T,
                      b_ih.reshape(1, 3 * H), W_hh.T, b_hh.reshape(1, 3 * H))
    return (h[:N], e_p[:E])


# speedup vs baseline: 2.8987x; 2.8987x over previous
import functools

import jax
import jax.numpy as jnp
from jax import lax
from jax.experimental import pallas as pl
from jax.experimental.pallas import tpu as pltpu
from jax.experimental.pallas import tpu_sc as plsc

N, E, D, DE, H = 10000, 320000, 128, 16, 128

NW = 32
CHUNK = 128
EDGES_PER_TILE = 10112
EP = NW * EDGES_PER_TILE
NCHUNKS = EDGES_PER_TILE // CHUNK
NP = 10240
RPT = NP // 16
DUMMY = N

BN = 1024
BE = 4096

_mesh = plsc.VectorSubcoreMesh(core_axis_name="c", subcore_axis_name="s")


@functools.partial(
    pl.kernel,
    out_type=jax.ShapeDtypeStruct((2 * NP, H), jnp.float32),
    mesh=_mesh,
    scratch_types=[
        pltpu.VMEM((CHUNK,), jnp.int32),
        pltpu.VMEM((CHUNK,), jnp.int32),
        pltpu.VMEM((CHUNK, H), jnp.float32),
        pltpu.VMEM_SHARED((NP, H), jnp.float32),
        pltpu.SemaphoreType.DMA,
    ],
)
def _sc_gather_scatter(h_hbm, src_hbm, dst_hbm, zer_hbm, out_hbm,
                       src_v, dst_v, rows_v, acc_sh, sem):
    cid = lax.axis_index("c")
    sid = lax.axis_index("s")
    wid = sid * 2 + cid
    rbase = sid * RPT
    pltpu.sync_copy(zer_hbm.at[pl.ds(rbase, RPT)], acc_sh.at[pl.ds(rbase, RPT)])
    plsc.subcore_barrier()

    ebase = wid * EDGES_PER_TILE

    def body(j, carry):
        off = ebase + j * CHUNK
        pltpu.sync_copy(src_hbm.at[pl.ds(off, CHUNK)], src_v)
        pltpu.sync_copy(dst_hbm.at[pl.ds(off, CHUNK)], dst_v)
        pltpu.async_copy(h_hbm.at[src_v], rows_v, sem).wait()
        pltpu.sync_copy(rows_v, acc_sh.at[dst_v], add=True)
        return carry

    lax.fori_loop(0, NCHUNKS, body, 0)
    plsc.subcore_barrier()
    pltpu.sync_copy(acc_sh.at[pl.ds(rbase, RPT)],
                    out_hbm.at[pl.ds(cid * NP + rbase, RPT)])


def _edge_body(ef_ref, WeT_ref, be_ref, Wm2T_ref, bm_ref, e_ref, e2_ref):
    e = (jnp.dot(ef_ref[...], WeT_ref[...],
                 preferred_element_type=jnp.float32) + be_ref[...])
    e_ref[...] = e
    e2_ref[...] = (jnp.dot(e, Wm2T_ref[...],
                           preferred_element_type=jnp.float32) + bm_ref[...])


def _init_body(nf_ref, WnT_ref, bn_ref, q0_ref, q1_ref, h0_ref, eb_ref):
    h0_ref[...] = (jnp.dot(nf_ref[...], WnT_ref[...],
                           preferred_element_type=jnp.float32) + bn_ref[...])
    eb_ref[...] = q0_ref[...] + q1_ref[...]


def _gru_body(p0_ref, p1_ref, eb_ref, h_ref, Wm1T_ref, W_ihT_ref, b_ih_ref,
              W_hhT_ref, b_hh_ref, hn_ref):
    agg = (jnp.dot(p0_ref[...] + p1_ref[...], Wm1T_ref[...],
                   preferred_element_type=jnp.float32) + eb_ref[...])
    gi = jnp.dot(agg, W_ihT_ref[...],
                 preferred_element_type=jnp.float32) + b_ih_ref[...]
    gh = jnp.dot(h_ref[...], W_hhT_ref[...],
                 preferred_element_type=jnp.float32) + b_hh_ref[...]
    r = jax.nn.sigmoid(gi[:, :H] + gh[:, :H])
    z = jax.nn.sigmoid(gi[:, H:2 * H] + gh[:, H:2 * H])
    n = jnp.tanh(gi[:, 2 * H:] + r * gh[:, 2 * H:])
    hn_ref[...] = (1.0 - z) * n + z * h_ref[...]


def _row_spec(b, w):
    return pl.BlockSpec((b, w), lambda i: (i, 0))


def _full_spec(r, c):
    return pl.BlockSpec((r, c), lambda i: (0, 0))


_edge_call = pl.pallas_call(
    _edge_body,
    grid=(EP // BE,),
    in_specs=[_row_spec(BE, DE), _full_spec(DE, H), _full_spec(1, H),
              _full_spec(H, H), _full_spec(1, H)],
    out_specs=[_row_spec(BE, H), _row_spec(BE, H)],
    out_shape=[jax.ShapeDtypeStruct((EP, H), jnp.float32),
               jax.ShapeDtypeStruct((EP, H), jnp.float32)],
)

_init_call = pl.pallas_call(
    _init_body,
    grid=(NP // BN,),
    in_specs=[_row_spec(BN, D), _full_spec(D, H), _full_spec(1, H),
              _row_spec(BN, H), _row_spec(BN, H)],
    out_specs=[_row_spec(BN, H), _row_spec(BN, H)],
    out_shape=[jax.ShapeDtypeStruct((NP, H), jnp.float32),
               jax.ShapeDtypeStruct((NP, H), jnp.float32)],
)

_gru_call = pl.pallas_call(
    _gru_body,
    grid=(NP // BN,),
    in_specs=[_row_spec(BN, H), _row_spec(BN, H), _row_spec(BN, H),
              _row_spec(BN, H),
              _full_spec(H, H), _full_spec(H, 3 * H), _full_spec(1, 3 * H),
              _full_spec(H, 3 * H), _full_spec(1, 3 * H)],
    out_specs=_row_spec(BN, H),
    out_shape=jax.ShapeDtypeStruct((NP, H), jnp.float32),
)


def kernel(node_feat, edge_index, edge_feat, Wn, bn, We, be, Wm, bm,
           W_ih, b_ih, W_hh, b_hh):
    src = edge_index[0]
    dst = edge_index[1]
    pad = EP - E
    src_p = jnp.concatenate([src, jnp.zeros((pad,), jnp.int32)])
    dst_p = jnp.concatenate([dst, jnp.full((pad,), DUMMY, jnp.int32)])
    ef_p = jnp.concatenate([edge_feat, jnp.zeros((pad, DE), jnp.float32)])
    nf_p = jnp.concatenate([node_feat, jnp.zeros((NP - N, D), jnp.float32)])

    zer = jnp.zeros((NP, H), jnp.float32)
    iota_e = jnp.arange(EP, dtype=jnp.int32)

    e_p, e2_p = _edge_call(ef_p, We.T, be.reshape(1, H), Wm[:, H:].T,
                           bm.reshape(1, H))
    q = _sc_gather_scatter(e2_p, iota_e, dst_p, zer)
    h, ebias = _init_call(nf_p, Wn.T, bn.reshape(1, H), q[:NP], q[NP:])
    for _ in range(3):
        p = _sc_gather_scatter(h, src_p, dst_p, zer)
        h = _gru_call(p[:NP], p[NP:], ebias, h, Wm[:, :H].T, W_ih.T,
                      b_ih.reshape(1, 3 * H), W_hh.T, b_hh.reshape(1, 3 * H))
    return (h[:N], e_p[:E])


# speedup vs baseline: 7.1328x; 2.4607x over previous
import functools

import jax
import jax.numpy as jnp
from jax import lax
from jax.experimental import pallas as pl
from jax.experimental.pallas import tpu as pltpu
from jax.experimental.pallas import tpu_sc as plsc

N, E, D, DE, H = 10000, 320000, 128, 16, 128

NW = 32
EPT = E // NW
CHUNK = 128
FULLCH = EPT // CHUNK
TAIL = EPT - FULLCH * CHUNK
DEPTH = 2
NP = 10112
RPT = NP // 16

BN = 1264
BE = 4000

_mesh = plsc.VectorSubcoreMesh(core_axis_name="c", subcore_axis_name="s")


@functools.partial(
    pl.kernel,
    out_type=jax.ShapeDtypeStruct((2 * NP, H), jnp.float32),
    mesh=_mesh,
    scratch_types=[
        pltpu.VMEM((EPT,), jnp.int32),
        pltpu.VMEM((CHUNK,), jnp.int32),
        pltpu.VMEM((CHUNK,), jnp.int32),
        pltpu.VMEM((TAIL,), jnp.int32),
        pltpu.VMEM((CHUNK, H), jnp.float32),
        pltpu.VMEM((CHUNK, H), jnp.float32),
        pltpu.VMEM((TAIL, H), jnp.float32),
        pltpu.VMEM_SHARED((NP, H), jnp.float32),
        pltpu.SemaphoreType.DMA,
        pltpu.SemaphoreType.DMA,
        pltpu.SemaphoreType.DMA,
        pltpu.SemaphoreType.DMA,
        pltpu.SemaphoreType.DMA,
        pltpu.SemaphoreType.DMA,
    ],
)
def _sc_gather_scatter(tab_hbm, src_hbm, dst_hbm, zer_hbm, out_hbm,
                       src_v, dst0, dst1, dstt, rows0, rows1, rowst, acc_sh,
                       semg0, semg1, sems0, sems1, semi0, semi1):
    cid = lax.axis_index("c")
    sid = lax.axis_index("s")
    wid = sid * 2 + cid
    rbase = sid * RPT
    pltpu.sync_copy(src_hbm.at[wid], src_v)
    pltpu.sync_copy(zer_hbm.at[pl.ds(rbase, RPT)], acc_sh.at[pl.ds(rbase, RPT)])
    plsc.subcore_barrier()

    rows = (rows0, rows1)
    dstv = (dst0, dst1)
    semg = (semg0, semg1)
    sems = (sems0, sems1)
    semi = (semi0, semi1)
    my_dst = dst_hbm.at[wid]

    def fire_gather(j, b):
        pltpu.async_copy(tab_hbm.at[src_v.at[pl.ds(j * CHUNK, CHUNK)]],
                         rows[b], semg[b])

    def wait_gather(j, b):
        pltpu.make_async_copy(tab_hbm.at[src_v.at[pl.ds(j * CHUNK, CHUNK)]],
                              rows[b], semg[b]).wait()

    def fire_dst(j, b):
        pltpu.async_copy(my_dst.at[pl.ds(j * CHUNK, CHUNK)], dstv[b], semi[b])

    def wait_dst(j, b):
        pltpu.make_async_copy(my_dst.at[pl.ds(j * CHUNK, CHUNK)], dstv[b],
                              semi[b]).wait()

    def fire_scatter(j, b):
        pltpu.async_copy(rows[b], acc_sh.at[dstv[b]], sems[b], add=True)

    def wait_scatter(j, b):
        pltpu.make_async_copy(rows[b], acc_sh.at[dstv[b]], sems[b]).wait()

    def body(g, carry):
        for b in range(DEPTH):
            j = g * DEPTH + b

            @pl.when(j >= DEPTH)
            def _():
                wait_scatter(j - DEPTH, b)

            fire_gather(j, b)
            fire_dst(j, b)

            @pl.when(j >= DEPTH - 1)
            def _():
                b2 = (b + 1) % DEPTH
                wait_gather(j - (DEPTH - 1), b2)
                wait_dst(j - (DEPTH - 1), b2)
                fire_scatter(j - (DEPTH - 1), b2)

        return carry

    lax.fori_loop(0, FULLCH // DEPTH, body, 0)

    jl = FULLCH - 1
    wait_gather(jl, jl % DEPTH)
    wait_dst(jl, jl % DEPTH)
    fire_scatter(jl, jl % DEPTH)
    for j in (FULLCH - 2, FULLCH - 1):
        wait_scatter(j, j % DEPTH)

    pltpu.sync_copy(my_dst.at[pl.ds(FULLCH * CHUNK, TAIL)], dstt)
    pltpu.async_copy(tab_hbm.at[src_v.at[pl.ds(FULLCH * CHUNK, TAIL)]],
                     rowst, semg0).wait()
    pltpu.async_copy(rowst, acc_sh.at[dstt], sems0, add=True).wait()

    plsc.subcore_barrier()
    pltpu.sync_copy(acc_sh.at[pl.ds(rbase, RPT)],
                    out_hbm.at[pl.ds(cid * NP + rbase, RPT)])


def _edge_body(ef_ref, WeT_ref, be_ref, Wm2T_ref, bm_ref, e_ref, e2_ref):
    e = (jnp.dot(ef_ref[...], WeT_ref[...],
                 preferred_element_type=jnp.float32) + be_ref[...])
    e_ref[...] = e
    e2_ref[...] = (jnp.dot(e, Wm2T_ref[...],
                           preferred_element_type=jnp.float32) + bm_ref[...])


def _init_body(nf_ref, WnT_ref, bn_ref, q0_ref, q1_ref, h0_ref, eb_ref):
    h0_ref[...] = (jnp.dot(nf_ref[...], WnT_ref[...],
                           preferred_element_type=jnp.float32) + bn_ref[...])
    eb_ref[...] = q0_ref[...] + q1_ref[...]


def _gru_body(p0_ref, p1_ref, eb_ref, h_ref, Wm1T_ref, W_ihT_ref, b_ih_ref,
              W_hhT_ref, b_hh_ref, hn_ref):
    agg = (jnp.dot(p0_ref[...] + p1_ref[...], Wm1T_ref[...],
                   preferred_element_type=jnp.float32) + eb_ref[...])
    gi = jnp.dot(agg, W_ihT_ref[...],
                 preferred_element_type=jnp.float32) + b_ih_ref[...]
    gh = jnp.dot(h_ref[...], W_hhT_ref[...],
                 preferred_element_type=jnp.float32) + b_hh_ref[...]
    r = jax.nn.sigmoid(gi[:, :H] + gh[:, :H])
    z = jax.nn.sigmoid(gi[:, H:2 * H] + gh[:, H:2 * H])
    n = jnp.tanh(gi[:, 2 * H:] + r * gh[:, 2 * H:])
    hn_ref[...] = (1.0 - z) * n + z * h_ref[...]


def _row_spec(b, w):
    return pl.BlockSpec((b, w), lambda i: (i, 0))


def _row_spec_off(b, w, off):
    return pl.BlockSpec((b, w), lambda i, o=off: (i + o, 0))


def _full_spec(r, c):
    return pl.BlockSpec((r, c), lambda i: (0, 0))


_edge_call = pl.pallas_call(
    _edge_body,
    grid=(E // BE,),
    in_specs=[_row_spec(BE, DE), _full_spec(DE, H), _full_spec(1, H),
              _full_spec(H, H), _full_spec(1, H)],
    out_specs=[_row_spec(BE, H), _row_spec(BE, H)],
    out_shape=[jax.ShapeDtypeStruct((E, H), jnp.float32),
               jax.ShapeDtypeStruct((E, H), jnp.float32)],
)

_init_call = pl.pallas_call(
    _init_body,
    grid=(NP // BN,),
    in_specs=[_row_spec(BN, D), _full_spec(D, H), _full_spec(1, H),
              _row_spec(BN, H), _row_spec_off(BN, H, NP // BN)],
    out_specs=[_row_spec(BN, H), _row_spec(BN, H)],
    out_shape=[jax.ShapeDtypeStruct((N, H), jnp.float32),
               jax.ShapeDtypeStruct((N, H), jnp.float32)],
)

_gru_call = pl.pallas_call(
    _gru_body,
    grid=(NP // BN,),
    in_specs=[_row_spec(BN, H), _row_spec_off(BN, H, NP // BN),
              _row_spec(BN, H), _row_spec(BN, H),
              _full_spec(H, H), _full_spec(H, 3 * H), _full_spec(1, 3 * H),
              _full_spec(H, 3 * H), _full_spec(1, 3 * H)],
    out_specs=_row_spec(BN, H),
    out_shape=jax.ShapeDtypeStruct((N, H), jnp.float32),
)


def kernel(node_feat, edge_index, edge_feat, Wn, bn, We, be, Wm, bm,
           W_ih, b_ih, W_hh, b_hh):
    srcm = edge_index[0].reshape(NW, EPT)
    dstm = edge_index[1].reshape(NW, EPT)
    iom = jnp.arange(E, dtype=jnp.int32).reshape(NW, EPT)
    zer = jnp.zeros((NP, H), jnp.float32)

    e, e2 = _edge_call(edge_feat, We.T, be.reshape(1, H), Wm[:, H:].T,
                       bm.reshape(1, H))
    q = _sc_gather_scatter(e2, iom, dstm, zer)
    h, ebias = _init_call(node_feat, Wn.T, bn.reshape(1, H), q, q)
    for _ in range(3):
        p = _sc_gather_scatter(h, srcm, dstm, zer)
        h = _gru_call(p, p, ebias, h, Wm[:, :H].T, W_ih.T,
                      b_ih.reshape(1, 3 * H), W_hh.T, b_hh.reshape(1, 3 * H))
    return (h, e)


# speedup vs baseline: 7.1426x; 1.0014x over previous
import functools

import jax
import jax.numpy as jnp
from jax import lax
from jax.experimental import pallas as pl
from jax.experimental.pallas import tpu as pltpu
from jax.experimental.pallas import tpu_sc as plsc

N, E, D, DE, H = 10000, 320000, 128, 16, 128

NW = 32
EPT = E // NW
CHUNK = 128
FULLCH = EPT // CHUNK
TAIL = EPT - FULLCH * CHUNK
DEPTH = 2
NP = 10112
RPT = NP // 16

BN = 1264
BE = 4000

_mesh = plsc.VectorSubcoreMesh(core_axis_name="c", subcore_axis_name="s")


@functools.partial(
    pl.kernel,
    out_type=jax.ShapeDtypeStruct((2 * NP, H), jnp.float32),
    mesh=_mesh,
    scratch_types=[
        pltpu.VMEM((EPT,), jnp.int32),
        pltpu.VMEM((CHUNK,), jnp.int32),
        pltpu.VMEM((CHUNK,), jnp.int32),
        pltpu.VMEM((TAIL,), jnp.int32),
        pltpu.VMEM((CHUNK, H), jnp.float32),
        pltpu.VMEM((CHUNK, H), jnp.float32),
        pltpu.VMEM((TAIL, H), jnp.float32),
        pltpu.VMEM_SHARED((NP, H), jnp.float32),
        pltpu.SemaphoreType.DMA,
        pltpu.SemaphoreType.DMA,
        pltpu.SemaphoreType.DMA,
        pltpu.SemaphoreType.DMA,
        pltpu.SemaphoreType.DMA,
        pltpu.SemaphoreType.DMA,
    ],
)
def _sc_gather_scatter(tab_hbm, src_hbm, dst_hbm, zer_hbm, out_hbm,
                       src_v, dst0, dst1, dstt, rows0, rows1, rowst, acc_sh,
                       semg0, semg1, sems0, sems1, semi0, semi1):
    cid = lax.axis_index("c")
    sid = lax.axis_index("s")
    wid = sid * 2 + cid
    rbase = sid * RPT
    pltpu.sync_copy(src_hbm.at[wid], src_v)
    pltpu.sync_copy(zer_hbm.at[pl.ds(rbase, RPT)], acc_sh.at[pl.ds(rbase, RPT)])
    plsc.subcore_barrier()

    rows = (rows0, rows1)
    dstv = (dst0, dst1)
    semg = (semg0, semg1)
    sems = (sems0, sems1)
    semi = (semi0, semi1)
    my_dst = dst_hbm.at[wid]

    def fire_gather(j, b):
        pltpu.async_copy(tab_hbm.at[src_v.at[pl.ds(j * CHUNK, CHUNK)]],
                         rows[b], semg[b])

    def wait_gather(j, b):
        pltpu.make_async_copy(tab_hbm.at[src_v.at[pl.ds(j * CHUNK, CHUNK)]],
                              rows[b], semg[b]).wait()

    def fire_dst(j, b):
        pltpu.async_copy(my_dst.at[pl.ds(j * CHUNK, CHUNK)], dstv[b], semi[b])

    def wait_dst(j, b):
        pltpu.make_async_copy(my_dst.at[pl.ds(j * CHUNK, CHUNK)], dstv[b],
                              semi[b]).wait()

    def fire_scatter(j, b):
        pltpu.async_copy(rows[b], acc_sh.at[dstv[b]], sems[b], add=True)

    def wait_scatter(j, b):
        pltpu.make_async_copy(rows[b], acc_sh.at[dstv[b]], sems[b]).wait()

    def body(g, carry):
        for b in range(DEPTH):
            j = g * DEPTH + b

            @pl.when(j >= DEPTH)
            def _():
                wait_scatter(j - DEPTH, b)

            fire_gather(j, b)
            fire_dst(j, b)

            @pl.when(j >= DEPTH - 1)
            def _():
                b2 = (b + 1) % DEPTH
                wait_gather(j - (DEPTH - 1), b2)
                wait_dst(j - (DEPTH - 1), b2)
                fire_scatter(j - (DEPTH - 1), b2)

        return carry

    lax.fori_loop(0, FULLCH // DEPTH, body, 0)

    jl = FULLCH - 1
    wait_gather(jl, jl % DEPTH)
    wait_dst(jl, jl % DEPTH)
    fire_scatter(jl, jl % DEPTH)
    for j in (FULLCH - 2, FULLCH - 1):
        wait_scatter(j, j % DEPTH)

    pltpu.sync_copy(my_dst.at[pl.ds(FULLCH * CHUNK, TAIL)], dstt)
    pltpu.async_copy(tab_hbm.at[src_v.at[pl.ds(FULLCH * CHUNK, TAIL)]],
                     rowst, semg0).wait()
    pltpu.async_copy(rowst, acc_sh.at[dstt], sems0, add=True).wait()

    plsc.subcore_barrier()
    pltpu.sync_copy(acc_sh.at[pl.ds(rbase, RPT)],
                    out_hbm.at[pl.ds(cid * NP + rbase, RPT)])


def _edge_body(ef_ref, WeT_ref, be_ref, Wm2T_ref, bm_ref, e_ref, e2_ref):
    e = (jnp.dot(ef_ref[...], WeT_ref[...],
                 preferred_element_type=jnp.float32) + be_ref[...])
    e_ref[...] = e
    e2_ref[...] = (jnp.dot(e, Wm2T_ref[...],
                           preferred_element_type=jnp.float32) + bm_ref[...])


def _init_body(nf_ref, WnT_ref, bn_ref, h0_ref):
    h0_ref[...] = (jnp.dot(nf_ref[...], WnT_ref[...],
                           preferred_element_type=jnp.float32) + bn_ref[...])


def _gru_body(p0_ref, p1_ref, q0_ref, q1_ref, h_ref, Wm1T_ref, W_ihT_ref,
              b_ih_ref, W_hhT_ref, b_hh_ref, hn_ref):
    agg = (jnp.dot(p0_ref[...] + p1_ref[...], Wm1T_ref[...],
                   preferred_element_type=jnp.float32)
           + q0_ref[...] + q1_ref[...])
    gi = jnp.dot(agg, W_ihT_ref[...],
                 preferred_element_type=jnp.float32) + b_ih_ref[...]
    gh = jnp.dot(h_ref[...], W_hhT_ref[...],
                 preferred_element_type=jnp.float32) + b_hh_ref[...]
    r = jax.nn.sigmoid(gi[:, :H] + gh[:, :H])
    z = jax.nn.sigmoid(gi[:, H:2 * H] + gh[:, H:2 * H])
    n = jnp.tanh(gi[:, 2 * H:] + r * gh[:, 2 * H:])
    hn_ref[...] = (1.0 - z) * n + z * h_ref[...]


def _row_spec(b, w):
    return pl.BlockSpec((b, w), lambda i: (i, 0))


def _row_spec_off(b, w, off):
    return pl.BlockSpec((b, w), lambda i, o=off: (i + o, 0))


def _full_spec(r, c):
    return pl.BlockSpec((r, c), lambda i: (0, 0))


_edge_call = pl.pallas_call(
    _edge_body,
    grid=(E // BE,),
    in_specs=[_row_spec(BE, DE), _full_spec(DE, H), _full_spec(1, H),
              _full_spec(H, H), _full_spec(1, H)],
    out_specs=[_row_spec(BE, H), _row_spec(BE, H)],
    out_shape=[jax.ShapeDtypeStruct((E, H), jnp.float32),
               jax.ShapeDtypeStruct((E, H), jnp.float32)],
)

_init_call = pl.pallas_call(
    _init_body,
    grid=(NP // BN,),
    in_specs=[_row_spec(BN, D), _full_spec(D, H), _full_spec(1, H)],
    out_specs=_row_spec(BN, H),
    out_shape=jax.ShapeDtypeStruct((N, H), jnp.float32),
)

_gru_call = pl.pallas_call(
    _gru_body,
    grid=(NP // BN,),
    in_specs=[_row_spec(BN, H), _row_spec_off(BN, H, NP // BN),
              _row_spec(BN, H), _row_spec_off(BN, H, NP // BN),
              _row_spec(BN, H),
              _full_spec(H, H), _full_spec(H, 3 * H), _full_spec(1, 3 * H),
              _full_spec(H, 3 * H), _full_spec(1, 3 * H)],
    out_specs=_row_spec(BN, H),
    out_shape=jax.ShapeDtypeStruct((N, H), jnp.float32),
)


def kernel(node_feat, edge_index, edge_feat, Wn, bn, We, be, Wm, bm,
           W_ih, b_ih, W_hh, b_hh):
    srcm = edge_index[0].reshape(NW, EPT)
    dstm = edge_index[1].reshape(NW, EPT)
    iom = jnp.arange(E, dtype=jnp.int32).reshape(NW, EPT)
    zer = jnp.zeros((NP, H), jnp.float32)

    h = _init_call(node_feat, Wn.T, bn.reshape(1, H))
    p = _sc_gather_scatter(h, srcm, dstm, zer)
    e, e2 = _edge_call(edge_feat, We.T, be.reshape(1, H), Wm[:, H:].T,
                       bm.reshape(1, H))
    q = _sc_gather_scatter(e2, iom, dstm, zer)
    h = _gru_call(p, p, q, q, h, Wm[:, :H].T, W_ih.T,
                  b_ih.reshape(1, 3 * H), W_hh.T, b_hh.reshape(1, 3 * H))
    for _ in range(2):
        p = _sc_gather_scatter(h, srcm, dstm, zer)
        h = _gru_call(p, p, q, q, h, Wm[:, :H].T, W_ih.T,
                      b_ih.reshape(1, 3 * H), W_hh.T, b_hh.reshape(1, 3 * H))
    return (h, e)


# speedup vs baseline: 7.2018x; 1.0083x over previous
import functools

import jax
import jax.numpy as jnp
from jax import lax
from jax.experimental import pallas as pl
from jax.experimental.pallas import tpu as pltpu
from jax.experimental.pallas import tpu_sc as plsc

N, E, D, DE, H = 10000, 320000, 128, 16, 128

NW = 32
EPT = E // NW
CHUNK = 128
FULLCH = EPT // CHUNK
TAIL = EPT - FULLCH * CHUNK
DEPTH = 2
NP = 10112
RPT = NP // 16

BN = 1264
BE = 4000

_mesh = plsc.VectorSubcoreMesh(core_axis_name="c", subcore_axis_name="s")


@functools.partial(
    pl.kernel,
    out_type=jax.ShapeDtypeStruct((2 * NP, H), jnp.float32),
    mesh=_mesh,
    scratch_types=[
        pltpu.VMEM((EPT,), jnp.int32),
        pltpu.VMEM((CHUNK,), jnp.int32),
        pltpu.VMEM((CHUNK,), jnp.int32),
        pltpu.VMEM((TAIL,), jnp.int32),
        pltpu.VMEM((CHUNK, H), jnp.float32),
        pltpu.VMEM((CHUNK, H), jnp.float32),
        pltpu.VMEM((TAIL, H), jnp.float32),
        pltpu.VMEM_SHARED((NP, H), jnp.float32),
        pltpu.SemaphoreType.DMA,
        pltpu.SemaphoreType.DMA,
        pltpu.SemaphoreType.DMA,
        pltpu.SemaphoreType.DMA,
        pltpu.SemaphoreType.DMA,
        pltpu.SemaphoreType.DMA,
    ],
)
def _sc_gather_scatter(tab_hbm, src_hbm, dst_hbm, zer_hbm, out_hbm,
                       src_v, dst0, dst1, dstt, rows0, rows1, rowst, acc_sh,
                       semg0, semg1, sems0, sems1, semi0, semi1):
    cid = lax.axis_index("c")
    sid = lax.axis_index("s")
    wid = sid * 2 + cid
    rbase = sid * RPT
    ebase = wid * EPT
    pltpu.sync_copy(src_hbm.at[pl.ds(ebase, EPT)], src_v)
    pltpu.sync_copy(zer_hbm.at[pl.ds(rbase, RPT)], acc_sh.at[pl.ds(rbase, RPT)])
    plsc.subcore_barrier()

    rows = (rows0, rows1)
    dstv = (dst0, dst1)
    semg = (semg0, semg1)
    sems = (sems0, sems1)
    semi = (semi0, semi1)

    def fire_gather(j, b):
        pltpu.async_copy(tab_hbm.at[src_v.at[pl.ds(j * CHUNK, CHUNK)]],
                         rows[b], semg[b])

    def wait_gather(j, b):
        pltpu.make_async_copy(tab_hbm.at[src_v.at[pl.ds(j * CHUNK, CHUNK)]],
                              rows[b], semg[b]).wait()

    def fire_dst(j, b):
        pltpu.async_copy(dst_hbm.at[pl.ds(ebase + j * CHUNK, CHUNK)],
                         dstv[b], semi[b])

    def wait_dst(j, b):
        pltpu.make_async_copy(dst_hbm.at[pl.ds(ebase + j * CHUNK, CHUNK)],
                              dstv[b], semi[b]).wait()

    def fire_scatter(j, b):
        pltpu.async_copy(rows[b], acc_sh.at[dstv[b]], sems[b], add=True)

    def wait_scatter(j, b):
        pltpu.make_async_copy(rows[b], acc_sh.at[dstv[b]], sems[b]).wait()

    def body(g, carry):
        for b in range(DEPTH):
            j = g * DEPTH + b

            @pl.when(j >= DEPTH)
            def _():
                wait_scatter(j - DEPTH, b)

            fire_gather(j, b)
            fire_dst(j, b)

            @pl.when(j >= DEPTH - 1)
            def _():
                b2 = (b + 1) % DEPTH
                wait_gather(j - (DEPTH - 1), b2)
                wait_dst(j - (DEPTH - 1), b2)
                fire_scatter(j - (DEPTH - 1), b2)

        return carry

    lax.fori_loop(0, FULLCH // DEPTH, body, 0)

    jl = FULLCH - 1
    wait_gather(jl, jl % DEPTH)
    wait_dst(jl, jl % DEPTH)
    fire_scatter(jl, jl % DEPTH)
    for j in (FULLCH - 2, FULLCH - 1):
        wait_scatter(j, j % DEPTH)

    pltpu.sync_copy(dst_hbm.at[pl.ds(ebase + FULLCH * CHUNK, TAIL)], dstt)
    pltpu.async_copy(tab_hbm.at[src_v.at[pl.ds(FULLCH * CHUNK, TAIL)]],
                     rowst, semg0).wait()
    pltpu.async_copy(rowst, acc_sh.at[dstt], sems0, add=True).wait()

    plsc.subcore_barrier()
    pltpu.sync_copy(acc_sh.at[pl.ds(rbase, RPT)],
                    out_hbm.at[pl.ds(cid * NP + rbase, RPT)])


def _edge_body(ef_ref, WeT_ref, be_ref, Wm2T_ref, bm_ref, e_ref, e2_ref):
    e = (jnp.dot(ef_ref[...], WeT_ref[...],
                 preferred_element_type=jnp.float32) + be_ref[...])
    e_ref[...] = e
    e2_ref[...] = (jnp.dot(e, Wm2T_ref[...],
                           preferred_element_type=jnp.float32) + bm_ref[...])


def _init_body(nf_ref, WnT_ref, bn_ref, h0_ref):
    h0_ref[...] = (jnp.dot(nf_ref[...], WnT_ref[...],
                           preferred_element_type=jnp.float32) + bn_ref[...])


def _gru_body(p0_ref, p1_ref, q0_ref, q1_ref, h_ref, Wm1T_ref, W_ihT_ref,
              b_ih_ref, W_hhT_ref, b_hh_ref, hn_ref):
    agg = (jnp.dot(p0_ref[...] + p1_ref[...], Wm1T_ref[...],
                   preferred_element_type=jnp.float32)
           + q0_ref[...] + q1_ref[...])
    gi = jnp.dot(agg, W_ihT_ref[...],
                 preferred_element_type=jnp.float32) + b_ih_ref[...]
    gh = jnp.dot(h_ref[...], W_hhT_ref[...],
                 preferred_element_type=jnp.float32) + b_hh_ref[...]
    r = jax.nn.sigmoid(gi[:, :H] + gh[:, :H])
    z = jax.nn.sigmoid(gi[:, H:2 * H] + gh[:, H:2 * H])
    n = jnp.tanh(gi[:, 2 * H:] + r * gh[:, 2 * H:])
    hn_ref[...] = (1.0 - z) * n + z * h_ref[...]


def _row_spec(b, w):
    return pl.BlockSpec((b, w), lambda i: (i, 0))


def _row_spec_off(b, w, off):
    return pl.BlockSpec((b, w), lambda i, o=off: (i + o, 0))


def _full_spec(r, c):
    return pl.BlockSpec((r, c), lambda i: (0, 0))


_edge_call = pl.pallas_call(
    _edge_body,
    grid=(E // BE,),
    in_specs=[_row_spec(BE, DE), _full_spec(DE, H), _full_spec(1, H),
              _full_spec(H, H), _full_spec(1, H)],
    out_specs=[_row_spec(BE, H), _row_spec(BE, H)],
    out_shape=[jax.ShapeDtypeStruct((E, H), jnp.float32),
               jax.ShapeDtypeStruct((E, H), jnp.float32)],
)

_init_call = pl.pallas_call(
    _init_body,
    grid=(NP // BN,),
    in_specs=[_row_spec(BN, D), _full_spec(D, H), _full_spec(1, H)],
    out_specs=_row_spec(BN, H),
    out_shape=jax.ShapeDtypeStruct((N, H), jnp.float32),
)

_gru_call = pl.pallas_call(
    _gru_body,
    grid=(NP // BN,),
    in_specs=[_row_spec(BN, H), _row_spec_off(BN, H, NP // BN),
              _row_spec(BN, H), _row_spec_off(BN, H, NP // BN),
              _row_spec(BN, H),
              _full_spec(H, H), _full_spec(H, 3 * H), _full_spec(1, 3 * H),
              _full_spec(H, 3 * H), _full_spec(1, 3 * H)],
    out_specs=_row_spec(BN, H),
    out_shape=jax.ShapeDtypeStruct((N, H), jnp.float32),
)


def kernel(node_feat, edge_index, edge_feat, Wn, bn, We, be, Wm, bm,
           W_ih, b_ih, W_hh, b_hh):
    srcm = edge_index[0]
    dstm = edge_index[1]
    iom = jnp.arange(E, dtype=jnp.int32)
    zer = jnp.zeros((NP, H), jnp.float32)

    h = _init_call(node_feat, Wn.T, bn.reshape(1, H))
    p = _sc_gather_scatter(h, srcm, dstm, zer)
    e, e2 = _edge_call(edge_feat, We.T, be.reshape(1, H), Wm[:, H:].T,
                       bm.reshape(1, H))
    q = _sc_gather_scatter(e2, iom, dstm, zer)
    h = _gru_call(p, p, q, q, h, Wm[:, :H].T, W_ih.T,
                  b_ih.reshape(1, 3 * H), W_hh.T, b_hh.reshape(1, 3 * H))
    for _ in range(2):
        p = _sc_gather_scatter(h, srcm, dstm, zer)
        h = _gru_call(p, p, q, q, h, Wm[:, :H].T, W_ih.T,
                      b_ih.reshape(1, 3 * H), W_hh.T, b_hh.reshape(1, 3 * H))
    return (h, e)
